# baseline (speedup 1.0000x reference)
import functools
import math

import jax
import jax.numpy as jnp
from jax import lax
from jax.experimental import pallas as pl
from jax.experimental.pallas import tpu as pltpu
from jax.experimental.pallas import tpu_sc as plsc

B = 512
N_NODES = 128
TOTAL = B * N_NODES
E = 524288
D = 128

G_PER_BLK = 8
ROWS_PER_BLK = G_PER_BLK * N_NODES
N_BLK = B // G_PER_BLK
N_PAIRS = N_NODES * (N_NODES - 1) // 2


def _ln(t, g, b, eps=1e-5):
    m = jnp.mean(t, axis=-1, keepdims=True)
    v = jnp.mean((t - m) ** 2, axis=-1, keepdims=True)
    return (t - m) * jax.lax.rsqrt(v + eps) * g + b


def _dense_body(C_ref, emb_ref, W1_ref, b1_ref, g1_ref, bt1_ref, W2_ref,
                b2_ref, Ws1_ref, bs1_ref, Ws2_ref, bs2_ref, gn_ref, bn_ref,
                We1_ref, be1_ref, ge_ref, bte_ref, We2r_ref, be2_ref,
                eo_ref, xo_ref):
    f32 = jnp.float32
    C = C_ref[...]
    h = jnp.dot(C, emb_ref[...], preferred_element_type=f32)
    t = jnp.dot(h, W1_ref[...], preferred_element_type=f32) + b1_ref[...]
    t = _ln(t, g1_ref[...], bt1_ref[...])
    t = jnp.maximum(t, 0.0)
    h = jnp.dot(t, W2_ref[...], preferred_element_type=f32) + b2_ref[...]
    t = jnp.maximum(jnp.dot(h, Ws1_ref[...], preferred_element_type=f32)
                    + bs1_ref[...], 0.0)
    h = jnp.dot(t, Ws2_ref[...], preferred_element_type=f32) + bs2_ref[...]
    hx = _ln(h, gn_ref[...], bn_ref[...])

    inv_sqrt_d = 1.0 / math.sqrt(float(D))
    for g in range(G_PER_BLK):
        xr = hx[g * N_NODES:(g + 1) * N_NODES, :]
        gram = lax.dot_general(xr, xr, (((1,), (1,)), ((), ())),
                               preferred_element_type=f32) * inv_sqrt_d
        off = 0
        for i in range(N_NODES - 1):
            ln_i = N_NODES - 1 - i
            eo_ref[g, pl.ds(off, ln_i)] = gram[i, i + 1:]
            off += ln_i

    means = jnp.mean(hx.reshape(G_PER_BLK, N_NODES, D), axis=1)
    he = _ln(jnp.dot(means, We1_ref[...], preferred_element_type=f32)
             + be1_ref[...], ge_ref[...], bte_ref[...])
    he = jnp.maximum(he, 0.0)
    xo_ref[...] = (jnp.sum(he * We2r_ref[...], axis=-1, keepdims=True)
                   + be2_ref[0, 0])


def _dense(C, emb, W1, b1, g1, bt1, W2, b2, Ws1, bs1, Ws2, bs2, gn, bn,
           We1, be1, ge, bte, We2, be2, interpret=False):
    row = lambda v: v.reshape(1, D)
    full = lambda shp: pl.BlockSpec(shp, lambda i: (0, 0))
    eo, xo = pl.pallas_call(
        _dense_body,
        grid=(N_BLK,),
        in_specs=[
            pl.BlockSpec((ROWS_PER_BLK, D), lambda i: (i, 0)),
            full((D, D)), full((D, D)), full((1, D)), full((1, D)),
            full((1, D)), full((D, D)), full((1, D)), full((D, D)),
            full((1, D)), full((D, D)), full((1, D)), full((1, D)),
            full((1, D)), full((D, D)), full((1, D)), full((1, D)),
            full((1, D)), full((1, D)), full((1, 1)),
        ],
        out_specs=[
            pl.BlockSpec((G_PER_BLK, N_PAIRS), lambda i: (i, 0)),
            pl.BlockSpec((G_PER_BLK, 1), lambda i: (i, 0)),
        ],
        out_shape=[
            jax.ShapeDtypeStruct((B, N_PAIRS), jnp.float32),
            jax.ShapeDtypeStruct((B, 1), jnp.float32),
        ],
        interpret=interpret,
    )(C, emb, W1, row(b1), row(g1), row(bt1), W2, row(b2), Ws1, row(bs1),
      Ws2, row(bs2), row(gn), row(bn), We1, row(be1), row(ge), row(bte),
      We2.reshape(1, D), be2.reshape(1, 1))
    return eo, xo


NC = 2
NS = 16
EDGES_PER_T = E // NS
SELF_PER_T = TOTAL // NS
FLAT_N = EDGES_PER_T + SELF_PER_T
N_ROUNDS = 4
SLICE_NODES = TOTAL // (N_ROUNDS * NC)
SLICE_WORDS = SLICE_NODES * N_NODES
STRIPE_WORDS = SLICE_WORDS // NS
CHUNK = 128
N_CHUNKS = FLAT_N // CHUNK
ZBUF = 4096
DUMMY = SLICE_WORDS


def _sc_hist_body(xp_hbm, src_hbm, dst_hbm, c_hbm,
                  xp_v, flat_v, s_v, d_v, idx_v, ones_v, zero_v, spmem):
    core = lax.axis_index("c")
    sub = lax.axis_index("s")

    def fill_zero(i, _):
        zero_v[pl.ds(i * 16, 16)] = jnp.zeros((16,), jnp.float32)
        return _
    lax.fori_loop(0, ZBUF // 16, fill_zero, None)

    def fill_ones(i, _):
        ones_v[pl.ds(i * 16, 16)] = jnp.ones((16,), jnp.float32)
        return _
    lax.fori_loop(0, CHUNK // 16, fill_ones, None)

    pltpu.sync_copy(xp_hbm, xp_v)

    def lookup(n):
        w = plsc.load_gather(xp_v, [lax.shift_right_logical(n, 2)])
        sh = lax.shift_left(jnp.bitwise_and(n, 3), 3)
        return jnp.bitwise_and(lax.shift_right_logical(w, sh), 127)

    ebase = sub * EDGES_PER_T
    for ck in range(EDGES_PER_T // 2048):
        pltpu.sync_copy(src_hbm.at[pl.ds(ebase + ck * 2048, 2048)], s_v)
        pltpu.sync_copy(dst_hbm.at[pl.ds(ebase + ck * 2048, 2048)], d_v)

        def prec(i, _):
            sv = s_v[pl.ds(i * 16, 16)]
            dv = d_v[pl.ds(i * 16, 16)]
            flat_v[pl.ds(ck * 2048 + i * 16, 16)] = dv * N_NODES + lookup(sv)
            return _
        lax.fori_loop(0, 2048 // 16, prec, None)

    nbase = sub * SELF_PER_T
    lane = lax.iota(jnp.int32, 16)

    def prec_self(i, _):
        nv = nbase + i * 16 + lane
        flat_v[pl.ds(EDGES_PER_T + i * 16, 16)] = nv * N_NODES + lookup(nv)
        return _
    lax.fori_loop(0, SELF_PER_T // 16, prec_self, None)

    for r in range(N_ROUNDS):
        sid = r * NC + core
        lo = sid * SLICE_WORDS
        hi = lo + SLICE_WORDS

        for k in range(STRIPE_WORDS // ZBUF):
            pltpu.sync_copy(
                zero_v, spmem.at[pl.ds(sub * STRIPE_WORDS + k * ZBUF, ZBUF)])
        plsc.subcore_barrier()

        def scan_chunk(c, _):
            def sub_step(j, _):
                f = flat_v[pl.ds(c * CHUNK + j * 16, 16)]
                m = (f >= lo) & (f < hi)
                idx_v[pl.ds(j * 16, 16)] = jnp.where(m, f - lo, DUMMY)
                return _
            lax.fori_loop(0, CHUNK // 16, sub_step, None)
            pltpu.sync_copy(ones_v, spmem.at[idx_v], add=True)
            return _
        lax.fori_loop(0, N_CHUNKS, scan_chunk, None)
        plsc.subcore_barrier()

        pltpu.sync_copy(
            spmem.at[pl.ds(sub * STRIPE_WORDS, STRIPE_WORDS)],
            c_hbm.at[pl.ds(lo + sub * STRIPE_WORDS, STRIPE_WORDS)])
        plsc.subcore_barrier()


@jax.jit
def _sc_histogram(node_ids, src, dst):
    xq = node_ids.reshape(TOTAL // 4, 4)
    xp = (xq[:, 0] | (xq[:, 1] << 8) | (xq[:, 2] << 16) | (xq[:, 3] << 24))
    k = pl.kernel(
        _sc_hist_body,
        mesh=plsc.VectorSubcoreMesh(core_axis_name="c", subcore_axis_name="s"),
        out_type=jax.ShapeDtypeStruct((TOTAL * N_NODES,), jnp.float32),
        compiler_params=pltpu.CompilerParams(needs_layout_passes=False),
        scratch_types=[
            pltpu.VMEM((TOTAL // 4,), jnp.int32),
            pltpu.VMEM((FLAT_N,), jnp.int32),
            pltpu.VMEM((2048,), jnp.int32),
            pltpu.VMEM((2048,), jnp.int32),
            pltpu.VMEM((CHUNK,), jnp.int32),
            pltpu.VMEM((CHUNK,), jnp.float32),
            pltpu.VMEM((ZBUF,), jnp.float32),
            pltpu.VMEM_SHARED((SLICE_WORDS + CHUNK,), jnp.float32),
        ],
    )
    return k(xp, src, dst).reshape(TOTAL, N_NODES)


def _histogram(node_ids, src, dst):
    xsrc = node_ids[src]
    Cm = jnp.zeros((TOTAL, N_NODES), jnp.float32)
    Cm = Cm.at[dst, xsrc].add(1.0)
    Cm = Cm.at[jnp.arange(TOTAL, dtype=jnp.int32), node_ids].add(1.0)
    return Cm


def kernel(x, edge_index, ptr, emb, W1, b1, g1, bt1, W2, b2, Ws1, bs1,
           Ws2, bs2, gn, bn, We1, be1, ge, bte, We2, be2):
    del ptr
    node_ids = x.reshape(TOTAL)
    Cm = _sc_histogram(node_ids, edge_index[0], edge_index[1])
    eo, xo = _dense(Cm, emb, W1, b1, g1, bt1, W2, b2, Ws1, bs1, Ws2, bs2,
                    gn, bn, We1, be1, ge, bte, We2, be2)
    return jnp.concatenate([eo, xo], axis=1)


# speedup vs baseline: 1.0010x; 1.0010x over previous
import functools
import math

import jax
import jax.numpy as jnp
from jax import lax
from jax.experimental import pallas as pl
from jax.experimental.pallas import tpu as pltpu
from jax.experimental.pallas import tpu_sc as plsc

B = 512
N_NODES = 128
TOTAL = B * N_NODES
E = 524288
D = 128

G_PER_BLK = 8
ROWS_PER_BLK = G_PER_BLK * N_NODES
N_BLK = B // G_PER_BLK
N_PAIRS = N_NODES * (N_NODES - 1) // 2


def _ln(t, g, b, eps=1e-5):
    m = jnp.mean(t, axis=-1, keepdims=True)
    v = jnp.mean((t - m) ** 2, axis=-1, keepdims=True)
    return (t - m) * jax.lax.rsqrt(v + eps) * g + b


def _dense_body(C_ref, emb_ref, W1_ref, b1_ref, g1_ref, bt1_ref, W2_ref,
                b2_ref, Ws1_ref, bs1_ref, Ws2_ref, bs2_ref, gn_ref, bn_ref,
                We1_ref, be1_ref, ge_ref, bte_ref, We2r_ref, be2_ref,
                eo_ref, xo_ref):
    f32 = jnp.float32
    C = C_ref[...]
    h = jnp.dot(C, emb_ref[...], preferred_element_type=f32)
    t = jnp.dot(h, W1_ref[...], preferred_element_type=f32) + b1_ref[...]
    t = _ln(t, g1_ref[...], bt1_ref[...])
    t = jnp.maximum(t, 0.0)
    h = jnp.dot(t, W2_ref[...], preferred_element_type=f32) + b2_ref[...]
    t = jnp.maximum(jnp.dot(h, Ws1_ref[...], preferred_element_type=f32)
                    + bs1_ref[...], 0.0)
    h = jnp.dot(t, Ws2_ref[...], preferred_element_type=f32) + bs2_ref[...]
    hx = _ln(h, gn_ref[...], bn_ref[...])

    inv_sqrt_d = 1.0 / math.sqrt(float(D))
    for g in range(G_PER_BLK):
        xr = hx[g * N_NODES:(g + 1) * N_NODES, :]
        gram = lax.dot_general(xr, xr, (((1,), (1,)), ((), ())),
                               preferred_element_type=f32) * inv_sqrt_d
        off = 0
        for i in range(N_NODES - 1):
            ln_i = N_NODES - 1 - i
            eo_ref[g, pl.ds(off, ln_i)] = gram[i, i + 1:]
            off += ln_i

    means = jnp.mean(hx.reshape(G_PER_BLK, N_NODES, D), axis=1)
    he = _ln(jnp.dot(means, We1_ref[...], preferred_element_type=f32)
             + be1_ref[...], ge_ref[...], bte_ref[...])
    he = jnp.maximum(he, 0.0)
    xo_ref[...] = (jnp.sum(he * We2r_ref[...], axis=-1, keepdims=True)
                   + be2_ref[0, 0])


def _dense(C, emb, W1, b1, g1, bt1, W2, b2, Ws1, bs1, Ws2, bs2, gn, bn,
           We1, be1, ge, bte, We2, be2, interpret=False):
    row = lambda v: v.reshape(1, D)
    full = lambda shp: pl.BlockSpec(shp, lambda i: (0, 0))
    eo, xo = pl.pallas_call(
        _dense_body,
        grid=(N_BLK,),
        in_specs=[
            pl.BlockSpec((ROWS_PER_BLK, D), lambda i: (i, 0)),
            full((D, D)), full((D, D)), full((1, D)), full((1, D)),
            full((1, D)), full((D, D)), full((1, D)), full((D, D)),
            full((1, D)), full((D, D)), full((1, D)), full((1, D)),
            full((1, D)), full((D, D)), full((1, D)), full((1, D)),
            full((1, D)), full((1, D)), full((1, 1)),
        ],
        out_specs=[
            pl.BlockSpec((G_PER_BLK, N_PAIRS), lambda i: (i, 0)),
            pl.BlockSpec((G_PER_BLK, 1), lambda i: (i, 0)),
        ],
        out_shape=[
            jax.ShapeDtypeStruct((B, N_PAIRS), jnp.float32),
            jax.ShapeDtypeStruct((B, 1), jnp.float32),
        ],
        interpret=interpret,
    )(C, emb, W1, row(b1), row(g1), row(bt1), W2, row(b2), Ws1, row(bs1),
      Ws2, row(bs2), row(gn), row(bn), We1, row(be1), row(ge), row(bte),
      We2.reshape(1, D), be2.reshape(1, 1))
    return eo, xo


NC = 2
NS = 16
EDGES_PER_T = E // NS
SELF_PER_T = TOTAL // NS
FLAT_N = EDGES_PER_T + SELF_PER_T
N_ROUNDS = 4
SLICE_NODES = TOTAL // (N_ROUNDS * NC)
SLICE_WORDS = SLICE_NODES * N_NODES
STRIPE_WORDS = SLICE_WORDS // NS
CHUNK = 128
N_CHUNKS = FLAT_N // CHUNK
RING = 8
ZBUF = 4096
DUMMY = SLICE_WORDS


def _sc_hist_body(xp_hbm, src_hbm, dst_hbm, c_hbm,
                  xp_v, flat_v, s_v, d_v, idx_v, ones_v, zero_v, spmem, sem):
    core = lax.axis_index("c")
    sub = lax.axis_index("s")

    def fill_zero(i, _):
        zero_v[pl.ds(i * 16, 16)] = jnp.zeros((16,), jnp.float32)
        return _
    lax.fori_loop(0, ZBUF // 16, fill_zero, None)

    def fill_ones(i, _):
        ones_v[pl.ds(i * 16, 16)] = jnp.ones((16,), jnp.float32)
        return _
    lax.fori_loop(0, CHUNK // 16, fill_ones, None)

    pltpu.sync_copy(xp_hbm, xp_v)

    def lookup(n):
        w = plsc.load_gather(xp_v, [lax.shift_right_logical(n, 2)])
        sh = lax.shift_left(jnp.bitwise_and(n, 3), 3)
        return jnp.bitwise_and(lax.shift_right_logical(w, sh), 127)

    ebase = sub * EDGES_PER_T
    for ck in range(EDGES_PER_T // 2048):
        pltpu.sync_copy(src_hbm.at[pl.ds(ebase + ck * 2048, 2048)], s_v)
        pltpu.sync_copy(dst_hbm.at[pl.ds(ebase + ck * 2048, 2048)], d_v)

        def prec(i, _):
            sv = s_v[pl.ds(i * 16, 16)]
            dv = d_v[pl.ds(i * 16, 16)]
            flat_v[pl.ds(ck * 2048 + i * 16, 16)] = dv * N_NODES + lookup(sv)
            return _
        lax.fori_loop(0, 2048 // 16, prec, None)

    nbase = sub * SELF_PER_T
    lane = lax.iota(jnp.int32, 16)

    def prec_self(i, _):
        nv = nbase + i * 16 + lane
        flat_v[pl.ds(EDGES_PER_T + i * 16, 16)] = nv * N_NODES + lookup(nv)
        return _
    lax.fori_loop(0, SELF_PER_T // 16, prec_self, None)

    for r in range(N_ROUNDS):
        sid = r * NC + core
        lo = sid * SLICE_WORDS
        hi = lo + SLICE_WORDS

        for k in range(STRIPE_WORDS // ZBUF):
            pltpu.async_copy(
                zero_v, spmem.at[pl.ds(sub * STRIPE_WORDS + k * ZBUF, ZBUF)],
                sem)
        for k in range(STRIPE_WORDS // ZBUF):
            pltpu.make_async_copy(
                zero_v, spmem.at[pl.ds(sub * STRIPE_WORDS + k * ZBUF, ZBUF)],
                sem).wait()
        plsc.subcore_barrier()

        def fill(c, b):
            def sub_step(j, _):
                f = flat_v[pl.ds(c * CHUNK + j * 16, 16)]
                m = (f >= lo) & (f < hi)
                idx_v[b, pl.ds(j * 16, 16)] = jnp.where(m, f - lo, DUMMY)
                return _
            lax.fori_loop(0, CHUNK // 16, sub_step, None)

        def start(b):
            pltpu.async_copy(ones_v, spmem.at[idx_v.at[b]], sem, add=True)

        def drain(b):
            pltpu.make_async_copy(ones_v, spmem.at[idx_v.at[b]], sem).wait()

        for b in range(RING):
            fill(b, b)
            start(b)

        def group(g, _):
            for b in range(RING):
                drain(b)
                fill(g * RING + b, b)
                start(b)
            return _
        lax.fori_loop(1, N_CHUNKS // RING, group, None)
        for b in range(RING):
            drain(b)
        plsc.subcore_barrier()

        pltpu.sync_copy(
            spmem.at[pl.ds(sub * STRIPE_WORDS, STRIPE_WORDS)],
            c_hbm.at[pl.ds(lo + sub * STRIPE_WORDS, STRIPE_WORDS)])
        plsc.subcore_barrier()


@jax.jit
def _sc_histogram(node_ids, src, dst):
    xq = node_ids.reshape(TOTAL // 4, 4)
    xp = (xq[:, 0] | (xq[:, 1] << 8) | (xq[:, 2] << 16) | (xq[:, 3] << 24))
    k = pl.kernel(
        _sc_hist_body,
        mesh=plsc.VectorSubcoreMesh(core_axis_name="c", subcore_axis_name="s"),
        out_type=jax.ShapeDtypeStruct((TOTAL * N_NODES,), jnp.float32),
        compiler_params=pltpu.CompilerParams(needs_layout_passes=False),
        scratch_types=[
            pltpu.VMEM((TOTAL // 4,), jnp.int32),
            pltpu.VMEM((FLAT_N,), jnp.int32),
            pltpu.VMEM((2048,), jnp.int32),
            pltpu.VMEM((2048,), jnp.int32),
            pltpu.VMEM((RING, CHUNK), jnp.int32),
            pltpu.VMEM((CHUNK,), jnp.float32),
            pltpu.VMEM((ZBUF,), jnp.float32),
            pltpu.VMEM_SHARED((SLICE_WORDS + CHUNK,), jnp.float32),
            pltpu.SemaphoreType.DMA,
        ],
    )
    return k(xp, src, dst).reshape(TOTAL, N_NODES)


def _histogram(node_ids, src, dst):
    xsrc = node_ids[src]
    Cm = jnp.zeros((TOTAL, N_NODES), jnp.float32)
    Cm = Cm.at[dst, xsrc].add(1.0)
    Cm = Cm.at[jnp.arange(TOTAL, dtype=jnp.int32), node_ids].add(1.0)
    return Cm


def kernel(x, edge_index, ptr, emb, W1, b1, g1, bt1, W2, b2, Ws1, bs1,
           Ws2, bs2, gn, bn, We1, be1, ge, bte, We2, be2):
    del ptr
    node_ids = x.reshape(TOTAL)
    Cm = _sc_histogram(node_ids, edge_index[0], edge_index[1])
    eo, xo = _dense(Cm, emb, W1, b1, g1, bt1, W2, b2, Ws1, bs1, Ws2, bs2,
                    gn, bn, We1, be1, ge, bte, We2, be2)
    return jnp.concatenate([eo, xo], axis=1)


# speedup vs baseline: 6.9706x; 6.9636x over previous
import functools
import math

import jax
import jax.numpy as jnp
from jax import lax
from jax.experimental import pallas as pl
from jax.experimental.pallas import tpu as pltpu
from jax.experimental.pallas import tpu_sc as plsc

B = 512
N_NODES = 128
TOTAL = B * N_NODES
E = 524288
D = 128

G_PER_BLK = 8
ROWS_PER_BLK = G_PER_BLK * N_NODES
N_BLK = B // G_PER_BLK
N_PAIRS = N_NODES * (N_NODES - 1) // 2


def _ln(t, g, b, eps=1e-5):
    m = jnp.mean(t, axis=-1, keepdims=True)
    v = jnp.mean((t - m) ** 2, axis=-1, keepdims=True)
    return (t - m) * jax.lax.rsqrt(v + eps) * g + b


def _dense_body(C_ref, emb_ref, W1_ref, b1_ref, g1_ref, bt1_ref, W2_ref,
                b2_ref, Ws1_ref, bs1_ref, Ws2_ref, bs2_ref, gn_ref, bn_ref,
                We1_ref, be1_ref, ge_ref, bte_ref, We2r_ref, be2_ref,
                eo_ref, xo_ref):
    f32 = jnp.float32
    C = C_ref[...]
    h = jnp.dot(C, emb_ref[...], preferred_element_type=f32)
    t = jnp.dot(h, W1_ref[...], preferred_element_type=f32) + b1_ref[...]
    t = _ln(t, g1_ref[...], bt1_ref[...])
    t = jnp.maximum(t, 0.0)
    h = jnp.dot(t, W2_ref[...], preferred_element_type=f32) + b2_ref[...]
    t = jnp.maximum(jnp.dot(h, Ws1_ref[...], preferred_element_type=f32)
                    + bs1_ref[...], 0.0)
    h = jnp.dot(t, Ws2_ref[...], preferred_element_type=f32) + bs2_ref[...]
    hx = _ln(h, gn_ref[...], bn_ref[...])

    inv_sqrt_d = 1.0 / math.sqrt(float(D))
    for g in range(G_PER_BLK):
        xr = hx[g * N_NODES:(g + 1) * N_NODES, :]
        gram = lax.dot_general(xr, xr, (((1,), (1,)), ((), ())),
                               preferred_element_type=f32) * inv_sqrt_d
        off = 0
        for i in range(N_NODES - 1):
            ln_i = N_NODES - 1 - i
            eo_ref[g, pl.ds(off, ln_i)] = gram[i, i + 1:]
            off += ln_i

    means = jnp.mean(hx.reshape(G_PER_BLK, N_NODES, D), axis=1)
    he = _ln(jnp.dot(means, We1_ref[...], preferred_element_type=f32)
             + be1_ref[...], ge_ref[...], bte_ref[...])
    he = jnp.maximum(he, 0.0)
    xo_ref[...] = (jnp.sum(he * We2r_ref[...], axis=-1, keepdims=True)
                   + be2_ref[0, 0])


def _dense(C, emb, W1, b1, g1, bt1, W2, b2, Ws1, bs1, Ws2, bs2, gn, bn,
           We1, be1, ge, bte, We2, be2, interpret=False):
    row = lambda v: v.reshape(1, D)
    full = lambda shp: pl.BlockSpec(shp, lambda i: (0, 0))
    eo, xo = pl.pallas_call(
        _dense_body,
        grid=(N_BLK,),
        in_specs=[
            pl.BlockSpec((ROWS_PER_BLK, D), lambda i: (i, 0)),
            full((D, D)), full((D, D)), full((1, D)), full((1, D)),
            full((1, D)), full((D, D)), full((1, D)), full((D, D)),
            full((1, D)), full((D, D)), full((1, D)), full((1, D)),
            full((1, D)), full((D, D)), full((1, D)), full((1, D)),
            full((1, D)), full((1, D)), full((1, 1)),
        ],
        out_specs=[
            pl.BlockSpec((G_PER_BLK, N_PAIRS), lambda i: (i, 0)),
            pl.BlockSpec((G_PER_BLK, 1), lambda i: (i, 0)),
        ],
        out_shape=[
            jax.ShapeDtypeStruct((B, N_PAIRS), jnp.float32),
            jax.ShapeDtypeStruct((B, 1), jnp.float32),
        ],
        interpret=interpret,
    )(C, emb, W1, row(b1), row(g1), row(bt1), W2, row(b2), Ws1, row(bs1),
      Ws2, row(bs2), row(gn), row(bn), We1, row(be1), row(ge), row(bte),
      We2.reshape(1, D), be2.reshape(1, 1))
    return eo, xo


NC = 2
NS = 16
EDGES_PER_T = E // NS
SELF_PER_T = TOTAL // NS
FLAT_N = EDGES_PER_T + SELF_PER_T
N_ROUNDS = 4
SLICE_NODES = TOTAL // (N_ROUNDS * NC)
SLICE_WORDS = SLICE_NODES * N_NODES
STRIPE_WORDS = SLICE_WORDS // NS
CHUNK = 128
N_CHUNKS = FLAT_N // CHUNK
RING = 8
ZBUF = 4096
DUMMY = SLICE_WORDS


def _sc_hist_body(xp_hbm, src_hbm, dst_hbm, c_hbm,
                  xp_v, flat_v, s_v, d_v, idx_v, ones_v, zero_v, spmem, sem):
    core = lax.axis_index("c")
    sub = lax.axis_index("s")

    def fill_zero(i, _):
        zero_v[pl.ds(i * 16, 16)] = jnp.zeros((16,), jnp.float32)
        return _
    lax.fori_loop(0, ZBUF // 16, fill_zero, None)

    def fill_ones(i, _):
        ones_v[pl.ds(i * 16, 16)] = jnp.ones((16,), jnp.float32)
        return _
    lax.fori_loop(0, CHUNK // 16, fill_ones, None)

    pltpu.sync_copy(xp_hbm, xp_v)

    def lookup(n):
        w = plsc.load_gather(xp_v, [lax.shift_right_logical(n, 2)])
        sh = lax.shift_left(jnp.bitwise_and(n, 3), 3)
        return jnp.bitwise_and(lax.shift_right_logical(w, sh), 127)

    ebase = sub * EDGES_PER_T
    for ck in range(EDGES_PER_T // 2048):
        pltpu.sync_copy(src_hbm.at[pl.ds(ebase + ck * 2048, 2048)], s_v)
        pltpu.sync_copy(dst_hbm.at[pl.ds(ebase + ck * 2048, 2048)], d_v)

        def prec(i, _):
            sv = s_v[pl.ds(i * 16, 16)]
            dv = d_v[pl.ds(i * 16, 16)]
            flat_v[pl.ds(ck * 2048 + i * 16, 16)] = dv * N_NODES + lookup(sv)
            return _
        lax.fori_loop(0, 2048 // 16, prec, None)

    nbase = sub * SELF_PER_T
    lane = lax.iota(jnp.int32, 16)

    def prec_self(i, _):
        nv = nbase + i * 16 + lane
        flat_v[pl.ds(EDGES_PER_T + i * 16, 16)] = nv * N_NODES + lookup(nv)
        return _
    lax.fori_loop(0, SELF_PER_T // 16, prec_self, None)

    for r in range(N_ROUNDS):
        sid = r * NC + core
        lo = sid * SLICE_WORDS
        hi = lo + SLICE_WORDS

        for k in range(STRIPE_WORDS // ZBUF):
            pltpu.async_copy(
                zero_v, spmem.at[pl.ds(sub * STRIPE_WORDS + k * ZBUF, ZBUF)],
                sem)
        for k in range(STRIPE_WORDS // ZBUF):
            pltpu.make_async_copy(
                zero_v, spmem.at[pl.ds(sub * STRIPE_WORDS + k * ZBUF, ZBUF)],
                sem).wait()
        plsc.subcore_barrier()

        def fill(c, b):
            def sub_step(j, _):
                f = flat_v[pl.ds(c * CHUNK + j * 16, 16)]
                m = (f >= lo) & (f < hi)
                pad = DUMMY + jnp.bitwise_and(f, 127)
                idx_v[b, pl.ds(j * 16, 16)] = jnp.where(m, f - lo, pad)
                return _
            lax.fori_loop(0, CHUNK // 16, sub_step, None)

        def start(b):
            pltpu.async_copy(ones_v, spmem.at[idx_v.at[b]], sem, add=True)

        def drain(b):
            pltpu.make_async_copy(ones_v, spmem.at[idx_v.at[b]], sem).wait()

        for b in range(RING):
            fill(b, b)
            start(b)

        def group(g, _):
            for b in range(RING):
                drain(b)
                fill(g * RING + b, b)
                start(b)
            return _
        lax.fori_loop(1, N_CHUNKS // RING, group, None)
        for b in range(RING):
            drain(b)
        plsc.subcore_barrier()

        pltpu.sync_copy(
            spmem.at[pl.ds(sub * STRIPE_WORDS, STRIPE_WORDS)],
            c_hbm.at[pl.ds(lo + sub * STRIPE_WORDS, STRIPE_WORDS)])
        plsc.subcore_barrier()


@jax.jit
def _sc_histogram(node_ids, src, dst):
    xq = node_ids.reshape(TOTAL // 4, 4)
    xp = (xq[:, 0] | (xq[:, 1] << 8) | (xq[:, 2] << 16) | (xq[:, 3] << 24))
    k = pl.kernel(
        _sc_hist_body,
        mesh=plsc.VectorSubcoreMesh(core_axis_name="c", subcore_axis_name="s"),
        out_type=jax.ShapeDtypeStruct((TOTAL * N_NODES,), jnp.float32),
        compiler_params=pltpu.CompilerParams(needs_layout_passes=False),
        scratch_types=[
            pltpu.VMEM((TOTAL // 4,), jnp.int32),
            pltpu.VMEM((FLAT_N,), jnp.int32),
            pltpu.VMEM((2048,), jnp.int32),
            pltpu.VMEM((2048,), jnp.int32),
            pltpu.VMEM((RING, CHUNK), jnp.int32),
            pltpu.VMEM((CHUNK,), jnp.float32),
            pltpu.VMEM((ZBUF,), jnp.float32),
            pltpu.VMEM_SHARED((SLICE_WORDS + CHUNK,), jnp.float32),
            pltpu.SemaphoreType.DMA,
        ],
    )
    return k(xp, src, dst).reshape(TOTAL, N_NODES)


def _histogram(node_ids, src, dst):
    xsrc = node_ids[src]
    Cm = jnp.zeros((TOTAL, N_NODES), jnp.float32)
    Cm = Cm.at[dst, xsrc].add(1.0)
    Cm = Cm.at[jnp.arange(TOTAL, dtype=jnp.int32), node_ids].add(1.0)
    return Cm


def kernel(x, edge_index, ptr, emb, W1, b1, g1, bt1, W2, b2, Ws1, bs1,
           Ws2, bs2, gn, bn, We1, be1, ge, bte, We2, be2):
    del ptr
    node_ids = x.reshape(TOTAL)
    Cm = _sc_histogram(node_ids, edge_index[0], edge_index[1])
    eo, xo = _dense(Cm, emb, W1, b1, g1, bt1, W2, b2, Ws1, bs1, Ws2, bs2,
                    gn, bn, We1, be1, ge, bte, We2, be2)
    return jnp.concatenate([eo, xo], axis=1)


# speedup vs baseline: 7.3893x; 1.0601x over previous
import functools
import math

import jax
import jax.numpy as jnp
from jax import lax
from jax.experimental import pallas as pl
from jax.experimental.pallas import tpu as pltpu
from jax.experimental.pallas import tpu_sc as plsc

B = 512
N_NODES = 128
TOTAL = B * N_NODES
E = 524288
D = 128

G_PER_BLK = 8
ROWS_PER_BLK = G_PER_BLK * N_NODES
N_BLK = B // G_PER_BLK
N_PAIRS = N_NODES * (N_NODES - 1) // 2


def _ln(t, g, b, eps=1e-5):
    m = jnp.mean(t, axis=-1, keepdims=True)
    v = jnp.mean((t - m) ** 2, axis=-1, keepdims=True)
    return (t - m) * jax.lax.rsqrt(v + eps) * g + b


def _dense_body(C_ref, emb_ref, W1_ref, b1_ref, g1_ref, bt1_ref, W2_ref,
                b2_ref, Ws1_ref, bs1_ref, Ws2_ref, bs2_ref, gn_ref, bn_ref,
                We1_ref, be1_ref, ge_ref, bte_ref, We2r_ref, be2_ref,
                eo_ref, xo_ref):
    f32 = jnp.float32
    C = C_ref[...]
    h = jnp.dot(C, emb_ref[...], preferred_element_type=f32)
    t = jnp.dot(h, W1_ref[...], preferred_element_type=f32) + b1_ref[...]
    t = _ln(t, g1_ref[...], bt1_ref[...])
    t = jnp.maximum(t, 0.0)
    h = jnp.dot(t, W2_ref[...], preferred_element_type=f32) + b2_ref[...]
    t = jnp.maximum(jnp.dot(h, Ws1_ref[...], preferred_element_type=f32)
                    + bs1_ref[...], 0.0)
    h = jnp.dot(t, Ws2_ref[...], preferred_element_type=f32) + bs2_ref[...]
    hx = _ln(h, gn_ref[...], bn_ref[...])

    inv_sqrt_d = 1.0 / math.sqrt(float(D))
    for g in range(G_PER_BLK):
        xr = hx[g * N_NODES:(g + 1) * N_NODES, :]
        gram = lax.dot_general(xr, xr, (((1,), (1,)), ((), ())),
                               preferred_element_type=f32) * inv_sqrt_d
        off = 0
        for i in range(N_NODES - 1):
            ln_i = N_NODES - 1 - i
            eo_ref[g, pl.ds(off, ln_i)] = gram[i, i + 1:]
            off += ln_i

    means = jnp.mean(hx.reshape(G_PER_BLK, N_NODES, D), axis=1)
    he = _ln(jnp.dot(means, We1_ref[...], preferred_element_type=f32)
             + be1_ref[...], ge_ref[...], bte_ref[...])
    he = jnp.maximum(he, 0.0)
    xo_ref[...] = (jnp.sum(he * We2r_ref[...], axis=-1, keepdims=True)
                   + be2_ref[0, 0])


def _dense(C, emb, W1, b1, g1, bt1, W2, b2, Ws1, bs1, Ws2, bs2, gn, bn,
           We1, be1, ge, bte, We2, be2, interpret=False):
    n_graphs = C.shape[0] // N_NODES
    row = lambda v: v.reshape(1, D)
    full = lambda shp: pl.BlockSpec(shp, lambda i: (0, 0))
    eo, xo = pl.pallas_call(
        _dense_body,
        grid=(n_graphs // G_PER_BLK,),
        in_specs=[
            pl.BlockSpec((ROWS_PER_BLK, D), lambda i: (i, 0)),
            full((D, D)), full((D, D)), full((1, D)), full((1, D)),
            full((1, D)), full((D, D)), full((1, D)), full((D, D)),
            full((1, D)), full((D, D)), full((1, D)), full((1, D)),
            full((1, D)), full((D, D)), full((1, D)), full((1, D)),
            full((1, D)), full((1, D)), full((1, 1)),
        ],
        out_specs=[
            pl.BlockSpec((G_PER_BLK, N_PAIRS), lambda i: (i, 0)),
            pl.BlockSpec((G_PER_BLK, 1), lambda i: (i, 0)),
        ],
        out_shape=[
            jax.ShapeDtypeStruct((n_graphs, N_PAIRS), jnp.float32),
            jax.ShapeDtypeStruct((n_graphs, 1), jnp.float32),
        ],
        interpret=interpret,
    )(C, emb, W1, row(b1), row(g1), row(bt1), W2, row(b2), Ws1, row(bs1),
      Ws2, row(bs2), row(gn), row(bn), We1, row(be1), row(ge), row(bte),
      We2.reshape(1, D), be2.reshape(1, 1))
    return eo, xo


NC = 2
NS = 16
EDGES_PER_T = E // NS
SELF_PER_T = TOTAL // NS
FLAT_N = EDGES_PER_T + SELF_PER_T
N_ROUNDS = 4
SLICE_NODES = TOTAL // (N_ROUNDS * NC)
SLICE_WORDS = SLICE_NODES * N_NODES
STRIPE_WORDS = SLICE_WORDS // NS
CHUNK = 128
N_CHUNKS = FLAT_N // CHUNK
RING = 8
ZBUF = 4096
DUMMY = SLICE_WORDS


def _sc_hist_body(r0, nr, xp_hbm, src_hbm, dst_hbm, c_hbm,
                  xp_v, flat_v, s_v, d_v, idx_v, ones_v, zero_v, spmem, sem):
    core = lax.axis_index("c")
    sub = lax.axis_index("s")

    def fill_zero(i, _):
        zero_v[pl.ds(i * 16, 16)] = jnp.zeros((16,), jnp.float32)
        return _
    lax.fori_loop(0, ZBUF // 16, fill_zero, None)

    def fill_ones(i, _):
        ones_v[pl.ds(i * 16, 16)] = jnp.ones((16,), jnp.float32)
        return _
    lax.fori_loop(0, CHUNK // 16, fill_ones, None)

    pltpu.sync_copy(xp_hbm, xp_v)

    def lookup(n):
        w = plsc.load_gather(xp_v, [lax.shift_right_logical(n, 2)])
        sh = lax.shift_left(jnp.bitwise_and(n, 3), 3)
        return jnp.bitwise_and(lax.shift_right_logical(w, sh), 127)

    ebase = sub * EDGES_PER_T
    for ck in range(EDGES_PER_T // 2048):
        pltpu.sync_copy(src_hbm.at[pl.ds(ebase + ck * 2048, 2048)], s_v)
        pltpu.sync_copy(dst_hbm.at[pl.ds(ebase + ck * 2048, 2048)], d_v)

        def prec(i, _):
            sv = s_v[pl.ds(i * 16, 16)]
            dv = d_v[pl.ds(i * 16, 16)]
            flat_v[pl.ds(ck * 2048 + i * 16, 16)] = dv * N_NODES + lookup(sv)
            return _
        lax.fori_loop(0, 2048 // 16, prec, None)

    nbase = sub * SELF_PER_T
    lane = lax.iota(jnp.int32, 16)

    def prec_self(i, _):
        nv = nbase + i * 16 + lane
        flat_v[pl.ds(EDGES_PER_T + i * 16, 16)] = nv * N_NODES + lookup(nv)
        return _
    lax.fori_loop(0, SELF_PER_T // 16, prec_self, None)

    for r in range(r0, r0 + nr):
        sid = r * NC + core
        lo = sid * SLICE_WORDS
        hi = lo + SLICE_WORDS
        out_base = lo - r0 * NC * SLICE_WORDS

        for k in range(STRIPE_WORDS // ZBUF):
            pltpu.async_copy(
                zero_v, spmem.at[pl.ds(sub * STRIPE_WORDS + k * ZBUF, ZBUF)],
                sem)
        for k in range(STRIPE_WORDS // ZBUF):
            pltpu.make_async_copy(
                zero_v, spmem.at[pl.ds(sub * STRIPE_WORDS + k * ZBUF, ZBUF)],
                sem).wait()
        plsc.subcore_barrier()

        def fill(c, b):
            def sub_step(j, _):
                f = flat_v[pl.ds(c * CHUNK + j * 16, 16)]
                m = (f >= lo) & (f < hi)
                pad = DUMMY + jnp.bitwise_and(f, 127)
                idx_v[b, pl.ds(j * 16, 16)] = jnp.where(m, f - lo, pad)
                return _
            lax.fori_loop(0, CHUNK // 16, sub_step, None)

        def start(b):
            pltpu.async_copy(ones_v, spmem.at[idx_v.at[b]], sem, add=True)

        def drain(b):
            pltpu.make_async_copy(ones_v, spmem.at[idx_v.at[b]], sem).wait()

        for b in range(RING):
            fill(b, b)
            start(b)

        def group(g, _):
            for b in range(RING):
                drain(b)
                fill(g * RING + b, b)
                start(b)
            return _
        lax.fori_loop(1, N_CHUNKS // RING, group, None)
        for b in range(RING):
            drain(b)
        plsc.subcore_barrier()

        pltpu.sync_copy(
            spmem.at[pl.ds(sub * STRIPE_WORDS, STRIPE_WORDS)],
            c_hbm.at[pl.ds(out_base + sub * STRIPE_WORDS, STRIPE_WORDS)])
        plsc.subcore_barrier()


def _sc_histogram(xp, src, dst, r0, nr):
    rows = nr * NC * SLICE_NODES
    k = pl.kernel(
        functools.partial(_sc_hist_body, r0, nr),
        mesh=plsc.VectorSubcoreMesh(core_axis_name="c", subcore_axis_name="s"),
        out_type=jax.ShapeDtypeStruct((rows * N_NODES,), jnp.float32),
        compiler_params=pltpu.CompilerParams(needs_layout_passes=False),
        scratch_types=[
            pltpu.VMEM((TOTAL // 4,), jnp.int32),
            pltpu.VMEM((FLAT_N,), jnp.int32),
            pltpu.VMEM((2048,), jnp.int32),
            pltpu.VMEM((2048,), jnp.int32),
            pltpu.VMEM((RING, CHUNK), jnp.int32),
            pltpu.VMEM((CHUNK,), jnp.float32),
            pltpu.VMEM((ZBUF,), jnp.float32),
            pltpu.VMEM_SHARED((SLICE_WORDS + CHUNK,), jnp.float32),
            pltpu.SemaphoreType.DMA,
        ],
    )
    return k(xp, src, dst).reshape(rows, N_NODES)


def _pack_x(node_ids):
    xq = node_ids.reshape(TOTAL // 4, 4)
    return xq[:, 0] | (xq[:, 1] << 8) | (xq[:, 2] << 16) | (xq[:, 3] << 24)


def _histogram(node_ids, src, dst):
    xsrc = node_ids[src]
    Cm = jnp.zeros((TOTAL, N_NODES), jnp.float32)
    Cm = Cm.at[dst, xsrc].add(1.0)
    Cm = Cm.at[jnp.arange(TOTAL, dtype=jnp.int32), node_ids].add(1.0)
    return Cm


def kernel(x, edge_index, ptr, emb, W1, b1, g1, bt1, W2, b2, Ws1, bs1,
           Ws2, bs2, gn, bn, We1, be1, ge, bte, We2, be2):
    del ptr
    node_ids = x.reshape(TOTAL)
    xp = _pack_x(node_ids)
    src, dst = edge_index[0], edge_index[1]
    n_part = 2
    per = N_ROUNDS // n_part
    eos, xos = [], []
    for p in range(n_part):
        Cp = _sc_histogram(xp, src, dst, p * per, per)
        eo, xo = _dense(Cp, emb, W1, b1, g1, bt1, W2, b2, Ws1, bs1, Ws2,
                        bs2, gn, bn, We1, be1, ge, bte, We2, be2)
        eos.append(eo)
        xos.append(xo)
    return jnp.concatenate(
        [jnp.concatenate(eos, axis=0), jnp.concatenate(xos, axis=0)], axis=1)


# speedup vs baseline: 7.4552x; 1.0089x over previous
import functools
import math

import jax
import jax.numpy as jnp
from jax import lax
from jax.experimental import pallas as pl
from jax.experimental.pallas import tpu as pltpu
from jax.experimental.pallas import tpu_sc as plsc

B = 512
N_NODES = 128
TOTAL = B * N_NODES
E = 524288
D = 128

G_PER_BLK = 8
ROWS_PER_BLK = G_PER_BLK * N_NODES
N_BLK = B // G_PER_BLK
N_PAIRS = N_NODES * (N_NODES - 1) // 2


def _ln(t, g, b, eps=1e-5):
    m = jnp.mean(t, axis=-1, keepdims=True)
    v = jnp.mean((t - m) ** 2, axis=-1, keepdims=True)
    return (t - m) * jax.lax.rsqrt(v + eps) * g + b


def _dense_body(C_ref, emb_ref, W1_ref, b1_ref, g1_ref, bt1_ref, W2_ref,
                b2_ref, Ws1_ref, bs1_ref, Ws2_ref, bs2_ref, gn_ref, bn_ref,
                We1_ref, be1_ref, ge_ref, bte_ref, We2r_ref, be2_ref,
                eo_ref, xo_ref):
    f32 = jnp.float32
    C = C_ref[...]
    h = jnp.dot(C, emb_ref[...], preferred_element_type=f32)
    t = jnp.dot(h, W1_ref[...], preferred_element_type=f32) + b1_ref[...]
    t = _ln(t, g1_ref[...], bt1_ref[...])
    t = jnp.maximum(t, 0.0)
    h = jnp.dot(t, W2_ref[...], preferred_element_type=f32) + b2_ref[...]
    t = jnp.maximum(jnp.dot(h, Ws1_ref[...], preferred_element_type=f32)
                    + bs1_ref[...], 0.0)
    h = jnp.dot(t, Ws2_ref[...], preferred_element_type=f32) + bs2_ref[...]
    hx = _ln(h, gn_ref[...], bn_ref[...])

    inv_sqrt_d = 1.0 / math.sqrt(float(D))
    for g in range(G_PER_BLK):
        xr = hx[g * N_NODES:(g + 1) * N_NODES, :]
        gram = lax.dot_general(xr, xr, (((1,), (1,)), ((), ())),
                               preferred_element_type=f32) * inv_sqrt_d
        off = 0
        for i in range(N_NODES - 1):
            ln_i = N_NODES - 1 - i
            eo_ref[g, pl.ds(off, ln_i)] = gram[i, i + 1:]
            off += ln_i

    means = jnp.mean(hx.reshape(G_PER_BLK, N_NODES, D), axis=1)
    he = _ln(jnp.dot(means, We1_ref[...], preferred_element_type=f32)
             + be1_ref[...], ge_ref[...], bte_ref[...])
    he = jnp.maximum(he, 0.0)
    xo_ref[...] = (jnp.sum(he * We2r_ref[...], axis=-1, keepdims=True)
                   + be2_ref[0, 0])


def _dense(C, emb, W1, b1, g1, bt1, W2, b2, Ws1, bs1, Ws2, bs2, gn, bn,
           We1, be1, ge, bte, We2, be2, interpret=False):
    n_graphs = C.shape[0] // N_NODES
    row = lambda v: v.reshape(1, D)
    full = lambda shp: pl.BlockSpec(shp, lambda i: (0, 0))
    eo, xo = pl.pallas_call(
        _dense_body,
        grid=(n_graphs // G_PER_BLK,),
        in_specs=[
            pl.BlockSpec((ROWS_PER_BLK, D), lambda i: (i, 0)),
            full((D, D)), full((D, D)), full((1, D)), full((1, D)),
            full((1, D)), full((D, D)), full((1, D)), full((D, D)),
            full((1, D)), full((D, D)), full((1, D)), full((1, D)),
            full((1, D)), full((D, D)), full((1, D)), full((1, D)),
            full((1, D)), full((1, D)), full((1, 1)),
        ],
        out_specs=[
            pl.BlockSpec((G_PER_BLK, N_PAIRS), lambda i: (i, 0)),
            pl.BlockSpec((G_PER_BLK, 1), lambda i: (i, 0)),
        ],
        out_shape=[
            jax.ShapeDtypeStruct((n_graphs, N_PAIRS), jnp.float32),
            jax.ShapeDtypeStruct((n_graphs, 1), jnp.float32),
        ],
        interpret=interpret,
    )(C, emb, W1, row(b1), row(g1), row(bt1), W2, row(b2), Ws1, row(bs1),
      Ws2, row(bs2), row(gn), row(bn), We1, row(be1), row(ge), row(bte),
      We2.reshape(1, D), be2.reshape(1, 1))
    return eo, xo


NC = 2
NS = 16
EDGES_PER_T = E // NS
SELF_PER_T = TOTAL // NS
FLAT_N = EDGES_PER_T + SELF_PER_T
N_ROUNDS = 4
SLICE_NODES = TOTAL // (N_ROUNDS * NC)
SLICE_WORDS = SLICE_NODES * N_NODES
STRIPE_WORDS = SLICE_WORDS // NS
CHUNK = 128
N_CHUNKS = FLAT_N // CHUNK
RING = 8
ZBUF = 4096
DUMMY = SLICE_WORDS


def _sc_hist_body(r0, nr, xp_hbm, src_hbm, dst_hbm, c_hbm,
                  xp_v, flat_v, s_v, d_v, idx_v, ones_v, zero_v, spmem, sem):
    core = lax.axis_index("c")
    sub = lax.axis_index("s")

    def fill_zero(i, _):
        zero_v[pl.ds(i * 16, 16)] = jnp.zeros((16,), jnp.float32)
        return _
    lax.fori_loop(0, ZBUF // 16, fill_zero, None)

    def fill_ones(i, _):
        ones_v[pl.ds(i * 16, 16)] = jnp.ones((16,), jnp.float32)
        return _
    lax.fori_loop(0, CHUNK // 16, fill_ones, None)

    pltpu.sync_copy(xp_hbm, xp_v)

    def lookup(n):
        w = plsc.load_gather(xp_v, [lax.shift_right_logical(n, 2)])
        sh = lax.shift_left(jnp.bitwise_and(n, 3), 3)
        return jnp.bitwise_and(lax.shift_right_logical(w, sh), 127)

    ebase = sub * EDGES_PER_T
    for ck in range(EDGES_PER_T // 2048):
        pltpu.sync_copy(src_hbm.at[pl.ds(ebase + ck * 2048, 2048)], s_v)
        pltpu.sync_copy(dst_hbm.at[pl.ds(ebase + ck * 2048, 2048)], d_v)

        def prec(i, _):
            for j in range(4):
                sv = s_v[pl.ds((i * 4 + j) * 16, 16)]
                dv = d_v[pl.ds((i * 4 + j) * 16, 16)]
                flat_v[pl.ds(ck * 2048 + (i * 4 + j) * 16, 16)] = (
                    dv * N_NODES + lookup(sv))
            return _
        lax.fori_loop(0, 2048 // 64, prec, None)

    nbase = sub * SELF_PER_T
    lane = lax.iota(jnp.int32, 16)

    def prec_self(i, _):
        for j in range(4):
            nv = nbase + (i * 4 + j) * 16 + lane
            flat_v[pl.ds(EDGES_PER_T + (i * 4 + j) * 16, 16)] = (
                nv * N_NODES + lookup(nv))
        return _
    lax.fori_loop(0, SELF_PER_T // 64, prec_self, None)

    for r in range(r0, r0 + nr):
        sid = r * NC + core
        lo = sid * SLICE_WORDS
        hi = lo + SLICE_WORDS
        out_base = lo - r0 * NC * SLICE_WORDS

        for k in range(STRIPE_WORDS // ZBUF):
            pltpu.async_copy(
                zero_v, spmem.at[pl.ds(sub * STRIPE_WORDS + k * ZBUF, ZBUF)],
                sem)
        for k in range(STRIPE_WORDS // ZBUF):
            pltpu.make_async_copy(
                zero_v, spmem.at[pl.ds(sub * STRIPE_WORDS + k * ZBUF, ZBUF)],
                sem).wait()
        plsc.subcore_barrier()

        def fill(c, b):
            for j in range(CHUNK // 16):
                f = flat_v[pl.ds(c * CHUNK + j * 16, 16)]
                m = (f >= lo) & (f < hi)
                pad = DUMMY + jnp.bitwise_and(f, 127)
                idx_v[b, pl.ds(j * 16, 16)] = jnp.where(m, f - lo, pad)

        def start(b):
            pltpu.async_copy(ones_v, spmem.at[idx_v.at[b]], sem, add=True)

        def drain(b):
            pltpu.make_async_copy(ones_v, spmem.at[idx_v.at[b]], sem).wait()

        for b in range(RING):
            fill(b, b)
            start(b)

        def group(g, _):
            for b in range(RING):
                drain(b)
                fill(g * RING + b, b)
                start(b)
            return _
        lax.fori_loop(1, N_CHUNKS // RING, group, None)
        for b in range(RING):
            drain(b)
        plsc.subcore_barrier()

        pltpu.sync_copy(
            spmem.at[pl.ds(sub * STRIPE_WORDS, STRIPE_WORDS)],
            c_hbm.at[pl.ds(out_base + sub * STRIPE_WORDS, STRIPE_WORDS)])
        plsc.subcore_barrier()


def _sc_histogram(xp, src, dst, r0, nr):
    rows = nr * NC * SLICE_NODES
    k = pl.kernel(
        functools.partial(_sc_hist_body, r0, nr),
        mesh=plsc.VectorSubcoreMesh(core_axis_name="c", subcore_axis_name="s"),
        out_type=jax.ShapeDtypeStruct((rows * N_NODES,), jnp.float32),
        compiler_params=pltpu.CompilerParams(needs_layout_passes=False),
        scratch_types=[
            pltpu.VMEM((TOTAL // 4,), jnp.int32),
            pltpu.VMEM((FLAT_N,), jnp.int32),
            pltpu.VMEM((2048,), jnp.int32),
            pltpu.VMEM((2048,), jnp.int32),
            pltpu.VMEM((RING, CHUNK), jnp.int32),
            pltpu.VMEM((CHUNK,), jnp.float32),
            pltpu.VMEM((ZBUF,), jnp.float32),
            pltpu.VMEM_SHARED((SLICE_WORDS + CHUNK,), jnp.float32),
            pltpu.SemaphoreType.DMA,
        ],
    )
    return k(xp, src, dst).reshape(rows, N_NODES)


def _pack_x(node_ids):
    xq = node_ids.reshape(TOTAL // 4, 4)
    return xq[:, 0] | (xq[:, 1] << 8) | (xq[:, 2] << 16) | (xq[:, 3] << 24)


def _histogram(node_ids, src, dst):
    xsrc = node_ids[src]
    Cm = jnp.zeros((TOTAL, N_NODES), jnp.float32)
    Cm = Cm.at[dst, xsrc].add(1.0)
    Cm = Cm.at[jnp.arange(TOTAL, dtype=jnp.int32), node_ids].add(1.0)
    return Cm


def kernel(x, edge_index, ptr, emb, W1, b1, g1, bt1, W2, b2, Ws1, bs1,
           Ws2, bs2, gn, bn, We1, be1, ge, bte, We2, be2):
    del ptr
    node_ids = x.reshape(TOTAL)
    xp = _pack_x(node_ids)
    src, dst = edge_index[0], edge_index[1]
    n_part = 2
    per = N_ROUNDS // n_part
    eos, xos = [], []
    for p in range(n_part):
        Cp = _sc_histogram(xp, src, dst, p * per, per)
        eo, xo = _dense(Cp, emb, W1, b1, g1, bt1, W2, b2, Ws1, bs1, Ws2,
                        bs2, gn, bn, We1, be1, ge, bte, We2, be2)
        eos.append(eo)
        xos.append(xo)
    return jnp.concatenate(
        [jnp.concatenate(eos, axis=0), jnp.concatenate(xos, axis=0)], axis=1)


# speedup vs baseline: 7.6859x; 1.0309x over previous
import functools
import math

import jax
import jax.numpy as jnp
from jax import lax
from jax.experimental import pallas as pl
from jax.experimental.pallas import tpu as pltpu
from jax.experimental.pallas import tpu_sc as plsc

B = 512
N_NODES = 128
TOTAL = B * N_NODES
E = 524288
D = 128

G_PER_BLK = 8
ROWS_PER_BLK = G_PER_BLK * N_NODES
N_BLK = B // G_PER_BLK
N_PAIRS = N_NODES * (N_NODES - 1) // 2


def _ln(t, g, b, eps=1e-5):
    m = jnp.mean(t, axis=-1, keepdims=True)
    v = jnp.mean((t - m) ** 2, axis=-1, keepdims=True)
    return (t - m) * jax.lax.rsqrt(v + eps) * g + b


def _dense_body(C_ref, emb_ref, W1_ref, b1_ref, g1_ref, bt1_ref, W2_ref,
                b2_ref, Ws1_ref, bs1_ref, Ws2_ref, bs2_ref, gn_ref, bn_ref,
                We1_ref, be1_ref, ge_ref, bte_ref, We2r_ref, be2_ref,
                eo_ref):
    f32 = jnp.float32
    C = C_ref[...]
    h = jnp.dot(C, emb_ref[...], preferred_element_type=f32)
    t = jnp.dot(h, W1_ref[...], preferred_element_type=f32) + b1_ref[...]
    t = _ln(t, g1_ref[...], bt1_ref[...])
    t = jnp.maximum(t, 0.0)
    h = jnp.dot(t, W2_ref[...], preferred_element_type=f32) + b2_ref[...]
    t = jnp.maximum(jnp.dot(h, Ws1_ref[...], preferred_element_type=f32)
                    + bs1_ref[...], 0.0)
    h = jnp.dot(t, Ws2_ref[...], preferred_element_type=f32) + bs2_ref[...]
    hx = _ln(h, gn_ref[...], bn_ref[...])

    inv_sqrt_d = 1.0 / math.sqrt(float(D))
    for g in range(G_PER_BLK):
        xr = hx[g * N_NODES:(g + 1) * N_NODES, :]
        gram = lax.dot_general(xr, xr, (((1,), (1,)), ((), ())),
                               preferred_element_type=f32) * inv_sqrt_d
        off = 0
        for i in range(N_NODES - 1):
            ln_i = N_NODES - 1 - i
            eo_ref[g, pl.ds(off, ln_i)] = gram[i, i + 1:]
            off += ln_i

    means = jnp.mean(hx.reshape(G_PER_BLK, N_NODES, D), axis=1)
    he = _ln(jnp.dot(means, We1_ref[...], preferred_element_type=f32)
             + be1_ref[...], ge_ref[...], bte_ref[...])
    he = jnp.maximum(he, 0.0)
    eo_ref[:, pl.ds(N_PAIRS, 1)] = (
        jnp.sum(he * We2r_ref[...], axis=-1, keepdims=True) + be2_ref[0, 0])


def _dense(C, emb, W1, b1, g1, bt1, W2, b2, Ws1, bs1, Ws2, bs2, gn, bn,
           We1, be1, ge, bte, We2, be2, interpret=False):
    n_graphs = C.shape[0] // N_NODES
    row = lambda v: v.reshape(1, D)
    full = lambda shp: pl.BlockSpec(shp, lambda i: (0, 0))
    eo = pl.pallas_call(
        _dense_body,
        grid=(n_graphs // G_PER_BLK,),
        in_specs=[
            pl.BlockSpec((ROWS_PER_BLK, D), lambda i: (i, 0)),
            full((D, D)), full((D, D)), full((1, D)), full((1, D)),
            full((1, D)), full((D, D)), full((1, D)), full((D, D)),
            full((1, D)), full((D, D)), full((1, D)), full((1, D)),
            full((1, D)), full((D, D)), full((1, D)), full((1, D)),
            full((1, D)), full((1, D)), full((1, 1)),
        ],
        out_specs=pl.BlockSpec((G_PER_BLK, N_PAIRS + 1), lambda i: (i, 0)),
        out_shape=jax.ShapeDtypeStruct((n_graphs, N_PAIRS + 1), jnp.float32),
        interpret=interpret,
    )(C, emb, W1, row(b1), row(g1), row(bt1), W2, row(b2), Ws1, row(bs1),
      Ws2, row(bs2), row(gn), row(bn), We1, row(be1), row(ge), row(bte),
      We2.reshape(1, D), be2.reshape(1, 1))
    return eo


NC = 2
NS = 16
EDGES_PER_T = E // NS
SELF_PER_T = TOTAL // NS
FLAT_N = EDGES_PER_T + SELF_PER_T
N_ROUNDS = 4
SLICE_NODES = TOTAL // (N_ROUNDS * NC)
SLICE_WORDS = SLICE_NODES * N_NODES
STRIPE_WORDS = SLICE_WORDS // NS
CHUNK = 128
N_CHUNKS = FLAT_N // CHUNK
RING = 8
ZBUF = 4096
DUMMY = SLICE_WORDS


def _sc_hist_body(r0, nr, xp_hbm, ei_hbm, c_hbm,
                  xp_v, flat_v, s_v, d_v, idx_v, ones_v, zero_v, spmem, sem):
    core = lax.axis_index("c")
    sub = lax.axis_index("s")

    def fill_zero(i, _):
        zero_v[pl.ds(i * 16, 16)] = jnp.zeros((16,), jnp.float32)
        return _
    lax.fori_loop(0, ZBUF // 16, fill_zero, None)

    def fill_ones(i, _):
        ones_v[pl.ds(i * 16, 16)] = jnp.ones((16,), jnp.float32)
        return _
    lax.fori_loop(0, CHUNK // 16, fill_ones, None)

    pltpu.sync_copy(xp_hbm, xp_v)

    def lookup(n):
        w = plsc.load_gather(xp_v, [lax.shift_right_logical(n, 2)])
        sh = lax.shift_left(jnp.bitwise_and(n, 3), 3)
        return jnp.bitwise_and(lax.shift_right_logical(w, sh), 127)

    ebase = sub * EDGES_PER_T
    for ck in range(EDGES_PER_T // 2048):
        pltpu.sync_copy(ei_hbm.at[0, pl.ds(ebase + ck * 2048, 2048)], s_v)
        pltpu.sync_copy(ei_hbm.at[1, pl.ds(ebase + ck * 2048, 2048)], d_v)

        def prec(i, _):
            for j in range(4):
                sv = s_v[pl.ds((i * 4 + j) * 16, 16)]
                dv = d_v[pl.ds((i * 4 + j) * 16, 16)]
                flat_v[pl.ds(ck * 2048 + (i * 4 + j) * 16, 16)] = (
                    dv * N_NODES + lookup(sv))
            return _
        lax.fori_loop(0, 2048 // 64, prec, None)

    nbase = sub * SELF_PER_T
    lane = lax.iota(jnp.int32, 16)

    def prec_self(i, _):
        for j in range(4):
            nv = nbase + (i * 4 + j) * 16 + lane
            flat_v[pl.ds(EDGES_PER_T + (i * 4 + j) * 16, 16)] = (
                nv * N_NODES + lookup(nv))
        return _
    lax.fori_loop(0, SELF_PER_T // 64, prec_self, None)

    for r in range(r0, r0 + nr):
        sid = r * NC + core
        lo = sid * SLICE_WORDS
        hi = lo + SLICE_WORDS
        out_base = lo - r0 * NC * SLICE_WORDS

        for k in range(STRIPE_WORDS // ZBUF):
            pltpu.async_copy(
                zero_v, spmem.at[pl.ds(sub * STRIPE_WORDS + k * ZBUF, ZBUF)],
                sem)
        for k in range(STRIPE_WORDS // ZBUF):
            pltpu.make_async_copy(
                zero_v, spmem.at[pl.ds(sub * STRIPE_WORDS + k * ZBUF, ZBUF)],
                sem).wait()
        plsc.subcore_barrier()

        def fill(c, b):
            for j in range(CHUNK // 16):
                f = flat_v[pl.ds(c * CHUNK + j * 16, 16)]
                m = (f >= lo) & (f < hi)
                pad = DUMMY + jnp.bitwise_and(f, 127)
                idx_v[b, pl.ds(j * 16, 16)] = jnp.where(m, f - lo, pad)

        def start(b):
            pltpu.async_copy(ones_v, spmem.at[idx_v.at[b]], sem, add=True)

        def drain(b):
            pltpu.make_async_copy(ones_v, spmem.at[idx_v.at[b]], sem).wait()

        for b in range(RING):
            fill(b, b)
            start(b)

        def group(g, _):
            for b in range(RING):
                drain(b)
                fill(g * RING + b, b)
                start(b)
            return _
        lax.fori_loop(1, N_CHUNKS // RING, group, None)
        for b in range(RING):
            drain(b)
        plsc.subcore_barrier()

        pltpu.sync_copy(
            spmem.at[pl.ds(sub * STRIPE_WORDS, STRIPE_WORDS)],
            c_hbm.at[pl.ds(out_base + sub * STRIPE_WORDS, STRIPE_WORDS)])
        plsc.subcore_barrier()


def _sc_histogram(xp, edge_index, r0, nr):
    rows = nr * NC * SLICE_NODES
    k = pl.kernel(
        functools.partial(_sc_hist_body, r0, nr),
        mesh=plsc.VectorSubcoreMesh(core_axis_name="c", subcore_axis_name="s"),
        out_type=jax.ShapeDtypeStruct((rows * N_NODES,), jnp.float32),
        compiler_params=pltpu.CompilerParams(needs_layout_passes=False),
        scratch_types=[
            pltpu.VMEM((TOTAL // 4,), jnp.int32),
            pltpu.VMEM((FLAT_N,), jnp.int32),
            pltpu.VMEM((2048,), jnp.int32),
            pltpu.VMEM((2048,), jnp.int32),
            pltpu.VMEM((RING, CHUNK), jnp.int32),
            pltpu.VMEM((CHUNK,), jnp.float32),
            pltpu.VMEM((ZBUF,), jnp.float32),
            pltpu.VMEM_SHARED((SLICE_WORDS + CHUNK,), jnp.float32),
            pltpu.SemaphoreType.DMA,
        ],
    )
    return k(xp, edge_index).reshape(rows, N_NODES)


def _pack_x(node_ids):
    xq = node_ids.reshape(TOTAL // 4, 4)
    return xq[:, 0] | (xq[:, 1] << 8) | (xq[:, 2] << 16) | (xq[:, 3] << 24)


def _histogram(node_ids, src, dst):
    xsrc = node_ids[src]
    Cm = jnp.zeros((TOTAL, N_NODES), jnp.float32)
    Cm = Cm.at[dst, xsrc].add(1.0)
    Cm = Cm.at[jnp.arange(TOTAL, dtype=jnp.int32), node_ids].add(1.0)
    return Cm


def kernel(x, edge_index, ptr, emb, W1, b1, g1, bt1, W2, b2, Ws1, bs1,
           Ws2, bs2, gn, bn, We1, be1, ge, bte, We2, be2):
    del ptr
    node_ids = x.reshape(TOTAL)
    xp = _pack_x(node_ids)
    n_part = 2
    per = N_ROUNDS // n_part
    eos = []
    for p in range(n_part):
        Cp = _sc_histogram(xp, edge_index, p * per, per)
        eos.append(_dense(Cp, emb, W1, b1, g1, bt1, W2, b2, Ws1, bs1, Ws2,
                          bs2, gn, bn, We1, be1, ge, bte, We2, be2))
    return jnp.concatenate(eos, axis=0)
